# Initial kernel scaffold; baseline (speedup 1.0000x reference)
import jax
import jax.numpy as jnp
from jax.experimental import pallas as pl


def kernel(node_ids, emb_table):
    raise NotImplementedError("write your pallas kernel here")


# speedup vs baseline: 4.9485x; 4.9485x over previous
import functools

import jax
import jax.numpy as jnp
from jax import lax
from jax.experimental import pallas as pl
from jax.experimental.pallas import tpu as pltpu
from jax.experimental.pallas import tpu_sc as plsc

N_DIM = 32
ROW_W = 128
K = 16


def _make_gather(n_rows: int, n_nodes: int, n_dim: int):
    info = plsc.get_sparse_core_info()
    nc, ns = info.num_cores, info.num_subcores
    nw = nc * ns
    rows_per_w = n_rows // nw
    n_chunks = rows_per_w // K

    mesh = plsc.VectorSubcoreMesh(core_axis_name="c", subcore_axis_name="s")

    @functools.partial(
        pl.kernel,
        mesh=mesh,
        out_type=jax.ShapeDtypeStruct((n_rows * ROW_W, n_dim), jnp.float32),
        scratch_types=[
            pltpu.VMEM((K, ROW_W), jnp.int32),
            pltpu.VMEM((K * ROW_W, n_dim), jnp.float32),
            pltpu.SemaphoreType.DMA,
        ],
        compiler_params=pltpu.CompilerParams(use_tc_tiling_on_sc=False),
    )
    def gather_kernel(idx_hbm, table_hbm, out_hbm, idx_v, rows_v, sem):
        wid = lax.axis_index("s") * nc + lax.axis_index("c")
        w_row0 = wid * rows_per_w

        def chunk_body(g, carry):
            row0 = w_row0 + g * K
            pltpu.sync_copy(idx_hbm.at[pl.ds(row0, K)], idx_v)
            copies = []
            for j in range(K):
                copies.append(pltpu.async_copy(
                    table_hbm.at[idx_v.at[j]],
                    rows_v.at[pl.ds(j * ROW_W, ROW_W)],
                    sem,
                ))
            for c in copies:
                c.wait()
            pltpu.sync_copy(rows_v, out_hbm.at[pl.ds(row0 * ROW_W, K * ROW_W)])
            return carry

        lax.fori_loop(0, n_chunks, chunk_body, 0)

    return gather_kernel


def kernel(node_ids, emb_table):
    b, h = node_ids.shape
    n_nodes, n_dim = emb_table.shape
    total = b * h
    n_rows = total // ROW_W
    idx2d = node_ids.reshape(n_rows, ROW_W).astype(jnp.int32)
    out = _make_gather(n_rows, n_nodes, n_dim)(idx2d, emb_table)
    return out.reshape(b, h, n_dim)
